# Initial kernel scaffold; baseline (speedup 1.0000x reference)
import jax
import jax.numpy as jnp
from jax.experimental import pallas as pl


def kernel(input_features, node_neigh_index, prob_retained, W1, b1, g1, bt1, W2, b2, g2, bt2):
    raise NotImplementedError("write your pallas kernel here")


# speedup vs baseline: 2.1551x; 2.1551x over previous
import functools

import jax
import jax.numpy as jnp
from jax import lax
from jax.experimental import pallas as pl
from jax.experimental.pallas import tpu as pltpu
from jax.experimental.pallas import tpu_sc as plsc

_N, _K, _C = 10000, 32, 128
_G = 4
_R = _G * _K
_NCHUNK = _N // _G
_NW = 32
_LANES = 16


def _mlp_body(x_ref, w1_ref, b1_ref, g1_ref, bt1_ref,
              w2_ref, b2_ref, g2_ref, bt2_ref, o_ref):
    h = x_ref[...]
    for w_ref, b_ref, ga_ref, be_ref in (
        (w1_ref, b1_ref, g1_ref, bt1_ref),
        (w2_ref, b2_ref, g2_ref, bt2_ref),
    ):
        h = lax.dot_general(h, w_ref[...], (((1,), (1,)), ((), ())),
                            preferred_element_type=jnp.float32)
        h = h + b_ref[...]
        h = jnp.where(h > 0, h, 0.2 * h)
        mu = jnp.mean(h, axis=0, keepdims=True)
        var = jnp.mean((h - mu) ** 2, axis=0, keepdims=True)
        h = (h - mu) * lax.rsqrt(var + 1e-5) * ga_ref[...] + be_ref[...]
    o_ref[...] = h


def _mlp(x, W1, b1, g1, bt1, W2, b2, g2, bt2):
    return pl.pallas_call(
        _mlp_body,
        out_shape=jax.ShapeDtypeStruct((_N, _C), jnp.float32),
    )(x, W1, b1.reshape(1, _C), g1.reshape(1, _C), bt1.reshape(1, _C),
      W2, b2.reshape(1, _C), g2.reshape(1, _C), bt2.reshape(1, _C))


@functools.partial(
    pl.kernel,
    out_type=jax.ShapeDtypeStruct((_N, _C), jnp.float32),
    mesh=plsc.VectorSubcoreMesh(core_axis_name="c", subcore_axis_name="s"),
    scratch_types=[
        pltpu.VMEM((_R,), jnp.int32),
        pltpu.VMEM((_R, _C), jnp.float32),
        pltpu.VMEM((_G, _C), jnp.float32),
        pltpu.SemaphoreType.DMA,
    ],
)
def _gather_mean(f_hbm, idx_hbm, out_hbm, idx_v, rows_v, out_v, sem):
    wid = lax.axis_index("s") * 2 + lax.axis_index("c")
    nbase = _NCHUNK // _NW
    nextra = _NCHUNK % _NW
    my_n = nbase + jnp.where(wid < nextra, 1, 0)

    def body(i, carry):
        c = wid + i * _NW
        pltpu.sync_copy(idx_hbm.at[pl.ds(c * _R, _R)], idx_v)
        pltpu.async_copy(f_hbm.at[idx_v], rows_v, sem).wait()
        for g in range(_G):
            for cb in range(_C // _LANES):
                col = pl.ds(cb * _LANES, _LANES)
                acc = rows_v[g * _K, col]
                for r in range(1, _K):
                    acc = acc + rows_v[g * _K + r, col]
                out_v[g, col] = acc * (1.0 / _K)
        pltpu.sync_copy(out_v, out_hbm.at[pl.ds(c * _G, _G)])
        return carry

    lax.fori_loop(0, my_n, body, 0)


def kernel(input_features, node_neigh_index, prob_retained,
           W1, b1, g1, bt1, W2, b2, g2, bt2):
    del prob_retained
    f = _mlp(input_features, W1, b1, g1, bt1, W2, b2, g2, bt2)
    idx_flat = node_neigh_index.reshape(-1).astype(jnp.int32)
    node_update = _gather_mean(f, idx_flat)
    return (node_update, f)


# speedup vs baseline: 3.2372x; 1.5021x over previous
import functools

import jax
import jax.numpy as jnp
from jax import lax
from jax.experimental import pallas as pl
from jax.experimental.pallas import tpu as pltpu
from jax.experimental.pallas import tpu_sc as plsc

_N, _K, _C = 10000, 32, 128
_G = 4
_R = _G * _K
_NCHUNK = _N // _G
_NW = 32
_LANES = 16
_NBASE = _NCHUNK // _NW
_NEXTRA = _NCHUNK - _NBASE * _NW
_MAXG = 80


def _mlp_body(x_ref, w1_ref, b1_ref, g1_ref, bt1_ref,
              w2_ref, b2_ref, g2_ref, bt2_ref, o_ref):
    h = x_ref[...]
    for w_ref, b_ref, ga_ref, be_ref in (
        (w1_ref, b1_ref, g1_ref, bt1_ref),
        (w2_ref, b2_ref, g2_ref, bt2_ref),
    ):
        h = lax.dot_general(h, w_ref[...], (((1,), (1,)), ((), ())),
                            preferred_element_type=jnp.float32)
        h = h + b_ref[...]
        h = jnp.where(h > 0, h, 0.2 * h)
        mu = jnp.mean(h, axis=0, keepdims=True)
        var = jnp.mean((h - mu) ** 2, axis=0, keepdims=True)
        h = (h - mu) * lax.rsqrt(var + 1e-5) * ga_ref[...] + be_ref[...]
    o_ref[...] = h


def _mlp(x, W1, b1, g1, bt1, W2, b2, g2, bt2):
    return pl.pallas_call(
        _mlp_body,
        out_shape=jax.ShapeDtypeStruct((_N, _C), jnp.float32),
    )(x, W1, b1.reshape(1, _C), g1.reshape(1, _C), bt1.reshape(1, _C),
      W2, b2.reshape(1, _C), g2.reshape(1, _C), bt2.reshape(1, _C))


@functools.partial(
    pl.kernel,
    out_type=jax.ShapeDtypeStruct((_N, _C), jnp.float32),
    mesh=plsc.VectorSubcoreMesh(core_axis_name="c", subcore_axis_name="s"),
    scratch_types=[
        pltpu.VMEM(((_NBASE + 1) * _R,), jnp.int32),
        pltpu.VMEM((_R, _C), jnp.float32),
        pltpu.VMEM((_R, _C), jnp.float32),
        pltpu.VMEM((_G, _C), jnp.float32),
        pltpu.VMEM((_G, _C), jnp.float32),
        pltpu.SemaphoreType.DMA,
        pltpu.SemaphoreType.DMA,
        pltpu.SemaphoreType.DMA,
        pltpu.SemaphoreType.DMA,
    ],
)
def _gather_mean(f_hbm, idx_hbm, out_hbm, idx_v, rows0, rows1,
                 outs0, outs1, gsem0, gsem1, osem0, osem1):
    w = lax.axis_index("s") * 2 + lax.axis_index("c")
    start = _NBASE * w + jnp.maximum(w - (_NW - _NEXTRA), 0)
    n = jnp.where(w >= _NW - _NEXTRA, _NBASE + 1, _NBASE)

    pltpu.sync_copy(idx_hbm.at[pl.ds(start * _R, (_NBASE + 1) * _R)], idx_v)
    pltpu.async_copy(f_hbm.at[idx_v.at[pl.ds(0, _R)]], rows0, gsem0)
    pltpu.async_copy(f_hbm.at[idx_v.at[pl.ds(_R, _R)]], rows1, gsem1)

    def do_group(i, rows_b, outs_b, gsem_b, osem_b):
        j = jnp.minimum(i, n - 1)
        c = start + j
        dst = out_hbm.at[pl.ds(c * _G, _G)]
        pltpu.make_async_copy(
            f_hbm.at[idx_v.at[pl.ds(j * _R, _R)]], rows_b, gsem_b).wait()
        @pl.when(i >= 2)
        def _():
            pltpu.make_async_copy(outs_b, dst, osem_b).wait()
        for g in range(_G):
            for cb in range(_C // _LANES):
                col = pl.ds(cb * _LANES, _LANES)
                acc = rows_b[g * _K, col]
                for r in range(1, _K):
                    acc = acc + rows_b[g * _K + r, col]
                outs_b[g, col] = acc * (1.0 / _K)
        pltpu.async_copy(outs_b, dst, osem_b)
        @pl.when(i + 2 < _MAXG)
        def _():
            j2 = jnp.minimum(i + 2, n - 1)
            pltpu.async_copy(
                f_hbm.at[idx_v.at[pl.ds(j2 * _R, _R)]], rows_b, gsem_b)

    def body(it, carry):
        do_group(2 * it, rows0, outs0, gsem0, osem0)
        do_group(2 * it + 1, rows1, outs1, gsem1, osem1)
        return carry

    lax.fori_loop(0, _MAXG // 2, body, 0)

    tail = out_hbm.at[pl.ds((start + n - 1) * _G, _G)]
    pltpu.make_async_copy(outs0, tail, osem0).wait()
    pltpu.make_async_copy(outs1, tail, osem1).wait()


def kernel(input_features, node_neigh_index, prob_retained,
           W1, b1, g1, bt1, W2, b2, g2, bt2):
    del prob_retained
    f = _mlp(input_features, W1, b1, g1, bt1, W2, b2, g2, bt2)
    idx = node_neigh_index.reshape(-1).astype(jnp.int32)
    node_update = _gather_mean(f, idx)
    return (node_update, f)


# speedup vs baseline: 3.2654x; 1.0087x over previous
import functools

import jax
import jax.numpy as jnp
from jax import lax
from jax.experimental import pallas as pl
from jax.experimental.pallas import tpu as pltpu
from jax.experimental.pallas import tpu_sc as plsc

_N, _K, _C = 10000, 32, 128
_G = 4
_R = _G * _K
_NCHUNK = _N // _G
_NW = 32
_LANES = 16
_NBASE = _NCHUNK // _NW
_NEXTRA = _NCHUNK - _NBASE * _NW
_MAXG = 80


def _mlp_body(x_ref, w1_ref, b1_ref, g1_ref, bt1_ref,
              w2_ref, b2_ref, g2_ref, bt2_ref, o_ref):
    h = x_ref[...]
    for w_ref, b_ref, ga_ref, be_ref in (
        (w1_ref, b1_ref, g1_ref, bt1_ref),
        (w2_ref, b2_ref, g2_ref, bt2_ref),
    ):
        h = lax.dot_general(h, w_ref[...], (((1,), (1,)), ((), ())),
                            preferred_element_type=jnp.float32)
        h = h + b_ref[...]
        h = jnp.where(h > 0, h, 0.2 * h)
        mu = jnp.mean(h, axis=0, keepdims=True)
        var = jnp.mean((h - mu) ** 2, axis=0, keepdims=True)
        h = (h - mu) * lax.rsqrt(var + 1e-5) * ga_ref[...] + be_ref[...]
    o_ref[...] = h


def _mlp(x, W1, b1, g1, bt1, W2, b2, g2, bt2):
    return pl.pallas_call(
        _mlp_body,
        out_shape=jax.ShapeDtypeStruct((_N, _C), jnp.float32),
    )(x, W1, b1.reshape(1, _C), g1.reshape(1, _C), bt1.reshape(1, _C),
      W2, b2.reshape(1, _C), g2.reshape(1, _C), bt2.reshape(1, _C))


@functools.partial(
    pl.kernel,
    out_type=jax.ShapeDtypeStruct((_N, _C), jnp.float32),
    mesh=plsc.VectorSubcoreMesh(core_axis_name="c", subcore_axis_name="s"),
    scratch_types=[
        pltpu.VMEM(((_NBASE + 1) * _R,), jnp.int32),
        pltpu.VMEM((_R, _C), jnp.float32),
        pltpu.VMEM((_R, _C), jnp.float32),
        pltpu.VMEM((_G, _C), jnp.float32),
        pltpu.VMEM((_G, _C), jnp.float32),
        pltpu.SemaphoreType.DMA,
        pltpu.SemaphoreType.DMA,
        pltpu.SemaphoreType.DMA,
        pltpu.SemaphoreType.DMA,
        pltpu.SemaphoreType.DMA,
        pltpu.SemaphoreType.DMA,
    ],
)
def _gather_mean(f_hbm, idx_hbm, out_hbm, idx_v, rows0, rows1,
                 outs0, outs1, gsem0a, gsem0b, gsem1a, gsem1b,
                 osem0, osem1):
    w = lax.axis_index("s") * 2 + lax.axis_index("c")
    start = _NBASE * w + jnp.maximum(w - (_NW - _NEXTRA), 0)
    n = jnp.where(w >= _NW - _NEXTRA, _NBASE + 1, _NBASE)

    pltpu.sync_copy(idx_hbm.at[pl.ds(start * _R, (_NBASE + 1) * _R)], idx_v)

    _H = _R // 2

    def start_gather(j, rows_b, sa, sb):
        pltpu.async_copy(
            f_hbm.at[idx_v.at[pl.ds(j * _R, _H)]],
            rows_b.at[pl.ds(0, _H)], sa)
        pltpu.async_copy(
            f_hbm.at[idx_v.at[pl.ds(j * _R + _H, _H)]],
            rows_b.at[pl.ds(_H, _H)], sb)

    def wait_gather(j, rows_b, sa, sb):
        pltpu.make_async_copy(
            f_hbm.at[idx_v.at[pl.ds(j * _R, _H)]],
            rows_b.at[pl.ds(0, _H)], sa).wait()
        pltpu.make_async_copy(
            f_hbm.at[idx_v.at[pl.ds(j * _R + _H, _H)]],
            rows_b.at[pl.ds(_H, _H)], sb).wait()

    start_gather(jnp.int32(0), rows0, gsem0a, gsem0b)
    start_gather(jnp.int32(1), rows1, gsem1a, gsem1b)

    def do_group(i, rows_b, outs_b, gsem_a, gsem_b, osem_b):
        j = jnp.minimum(i, n - 1)
        c = start + j
        dst = out_hbm.at[pl.ds(c * _G, _G)]
        wait_gather(j, rows_b, gsem_a, gsem_b)
        @pl.when(i >= 2)
        def _():
            pltpu.make_async_copy(outs_b, dst, osem_b).wait()
        for g in range(_G):
            for cb in range(_C // _LANES):
                col = pl.ds(cb * _LANES, _LANES)
                acc = rows_b[g * _K, col]
                for r in range(1, _K):
                    acc = acc + rows_b[g * _K + r, col]
                outs_b[g, col] = acc * (1.0 / _K)
        pltpu.async_copy(outs_b, dst, osem_b)
        @pl.when(i + 2 < _MAXG)
        def _():
            j2 = jnp.minimum(i + 2, n - 1)
            start_gather(j2, rows_b, gsem_a, gsem_b)

    def body(it, carry):
        do_group(2 * it, rows0, outs0, gsem0a, gsem0b, osem0)
        do_group(2 * it + 1, rows1, outs1, gsem1a, gsem1b, osem1)
        return carry

    lax.fori_loop(0, _MAXG // 2, body, 0)

    tail = out_hbm.at[pl.ds((start + n - 1) * _G, _G)]
    pltpu.make_async_copy(outs0, tail, osem0).wait()
    pltpu.make_async_copy(outs1, tail, osem1).wait()


def kernel(input_features, node_neigh_index, prob_retained,
           W1, b1, g1, bt1, W2, b2, g2, bt2):
    del prob_retained
    f = _mlp(input_features, W1, b1, g1, bt1, W2, b2, g2, bt2)
    idx = node_neigh_index.reshape(-1).astype(jnp.int32)
    node_update = _gather_mean(f, idx)
    return (node_update, f)
